# Initial kernel scaffold; baseline (speedup 1.0000x reference)
import jax
import jax.numpy as jnp
from jax.experimental import pallas as pl


def kernel(Xi, Xv, X_interest, w1, wi1, w2, wi2, bias, W1, b1, g1, be1, W2, b2, g2, be2, W3, b3, g3, be3):
    raise NotImplementedError("write your pallas kernel here")


# speedup vs baseline: 70.9853x; 70.9853x over previous
import functools

import jax
import jax.numpy as jnp
from jax import lax
from jax.experimental import pallas as pl
from jax.experimental.pallas import tpu as pltpu
from jax.experimental.pallas import tpu_sc as plsc

B = 16384; F = 26; NI = 4; H = 50; V = 100000; D = 16
L1 = 300; L2 = 300; L3 = 300
NC, NS = 2, 16
NW = NC * NS
LANES = 16

RP_A = (B * F) // NW
CH_A = 512
NCH_A = RP_A // CH_A


def _sc_gather_a(w2f, w1f, gidx, xvf):
    mesh = plsc.VectorSubcoreMesh(core_axis_name="c", subcore_axis_name="s")

    @functools.partial(
        pl.kernel,
        out_type=(
            jax.ShapeDtypeStruct((B * F, D), jnp.float32),
            jax.ShapeDtypeStruct((B * F,), jnp.float32),
        ),
        mesh=mesh,
        compiler_params=pltpu.CompilerParams(
            needs_layout_passes=False, use_tc_tiling_on_sc=False),
        scratch_types=[
            pltpu.VMEM((CH_A,), jnp.int32),
            pltpu.VMEM((CH_A,), jnp.float32),
            pltpu.VMEM((CH_A, D), jnp.float32),
            pltpu.VMEM((CH_A,), jnp.float32),
            pltpu.SemaphoreType.DMA,
            pltpu.SemaphoreType.DMA,
        ],
    )
    def k(w2f_h, w1f_h, gidx_h, xvf_h, e2_o, e1_o, idx_v, xv_v, rows_v, vals_v, sem1, sem2):
        wid = lax.axis_index("s") * NC + lax.axis_index("c")
        p0 = wid * RP_A

        def chunk(c, _):
            p = p0 + c * CH_A
            pltpu.sync_copy(gidx_h.at[pl.ds(p, CH_A)], idx_v)
            pltpu.sync_copy(xvf_h.at[pl.ds(p, CH_A)], xv_v)
            cp1 = pltpu.async_copy(w2f_h.at[idx_v], rows_v, sem1)
            cp2 = pltpu.async_copy(w1f_h.at[idx_v], vals_v, sem2)
            cp1.wait()
            cp2.wait()

            def srow(jj, _):
                g = jj * LANES
                xvg = xv_v[pl.ds(g, LANES)]
                for kk in range(LANES):
                    rows_v[g + kk, :] = rows_v[g + kk, :] * xvg[kk]
                return 0

            lax.fori_loop(0, CH_A // LANES, srow, 0)

            def sval(q, _):
                sl = pl.ds(q * LANES, LANES)
                vals_v[sl] = vals_v[sl] * xv_v[sl]
                return 0

            lax.fori_loop(0, CH_A // LANES, sval, 0)
            pltpu.sync_copy(rows_v, e2_o.at[pl.ds(p, CH_A), :])
            pltpu.sync_copy(vals_v, e1_o.at[pl.ds(p, CH_A)])
            return 0

        lax.fori_loop(0, NCH_A, chunk, 0)

    return k(w2f, w1f, gidx, xvf)


WPI = NW // NI
BPW = B // WPI
SB = 16
NBLK = BPW // SB


def _sc_segsum_b(wi2f, wi1f, xt):
    mesh = plsc.VectorSubcoreMesh(core_axis_name="c", subcore_axis_name="s")

    @functools.partial(
        pl.kernel,
        out_type=(
            jax.ShapeDtypeStruct((NI * B, D), jnp.float32),
            jax.ShapeDtypeStruct((NI * B,), jnp.float32),
        ),
        mesh=mesh,
        compiler_params=pltpu.CompilerParams(
            needs_layout_passes=False, use_tc_tiling_on_sc=False),
        scratch_types=[
            pltpu.VMEM((V,), jnp.float32),
            pltpu.VMEM((SB * H,), jnp.int32),
            pltpu.VMEM((SB * H,), jnp.int32),
            pltpu.VMEM((SB * H, D), jnp.float32),
            pltpu.VMEM((SB, D), jnp.float32),
            pltpu.VMEM((SB,), jnp.float32),
            pltpu.VMEM((1, D), jnp.float32),
            pltpu.SemaphoreType.DMA,
        ],
    )
    def k(wi2f_h, wi1f_h, xt_h, ei2_o, ei1_o,
          wtab, slab, gixv, rows, ei2b, ei1b, prow, sem):
        wid = lax.axis_index("s") * NC + lax.axis_index("c")
        i = wid // WPI
        b0 = (wid % WPI) * BPW
        iV = i * V
        pltpu.sync_copy(wi1f_h.at[pl.ds(iV, V)], wtab)
        pltpu.sync_copy(wi2f_h.at[pl.ds(iV + V - 1, 1), :], prow)
        posbase = lax.iota(jnp.int32, LANES) * H

        def blk(kb, _):
            b = b0 + kb * SB
            pltpu.sync_copy(xt_h.at[pl.ds((i * B + b) * H, SB * H)], slab)

            def gfix(q, _):
                sl = pl.ds(q * LANES, LANES)
                gixv[sl] = slab[sl] + iV
                return 0

            lax.fori_loop(0, SB * H // LANES, gfix, 0)
            cp = pltpu.async_copy(wi2f_h.at[gixv], rows, sem)

            def hstep(h, carry):
                acc, pcnt = carry
                xv = plsc.load_gather(slab, [posbase + h])
                m = xv == (V - 1)
                val = plsc.load_gather(wtab, [xv])
                acc = acc + jnp.where(m, 0.0, val)
                pcnt = pcnt + jnp.where(m, 1.0, 0.0)
                return acc, pcnt

            z = jnp.zeros((LANES,), jnp.float32)
            acc1, pcnt = lax.fori_loop(0, H, hstep, (z, z))
            ei1b[:] = acc1
            cp.wait()

            def hstep2(h, accs):
                return tuple(accs[s] + rows[s * H + h, :] for s in range(SB))

            accs = lax.fori_loop(
                0, H, hstep2, tuple(jnp.zeros((D,), jnp.float32) for _ in range(SB)))
            pr = prow[0, :]
            for s in range(SB):
                ei2b[s, :] = accs[s] - pcnt[s] * pr
            pltpu.sync_copy(ei2b, ei2_o.at[pl.ds(i * B + b, SB), :])
            pltpu.sync_copy(ei1b, ei1_o.at[pl.ds(i * B + b, SB)])
            return 0

        lax.fori_loop(0, NBLK, blk, 0)

    return k(wi2f, wi1f, xt)


BT = 512
GT = B // BT


def _k1_body(e2_ref, ei2_ref, ei1_ref, e1_ref, W1_ref, b1_ref, M_ref,
             h1_ref, sp_ref, ssp_ref, part_ref):
    x = e2_ref[...]
    h = jnp.dot(x, W1_ref[...], preferred_element_type=jnp.float32) + b1_ref[...]
    h1_ref[...] = h
    sp_ref[0, 0, :] = jnp.sum(h, 0)
    ssp_ref[0, 0, :] = jnp.sum(h * h, 0)
    Mm = M_ref[...]
    fmv = jnp.dot(x, Mm, preferred_element_type=jnp.float32)
    fmss = jnp.dot(x * x, Mm, preferred_element_type=jnp.float32)
    ei2 = ei2_ref[...]
    s_tot = fmv + jnp.sum(ei2, 0)
    sq_tot = fmss + jnp.sum(ei2 * ei2, 0)
    fm2 = 0.5 * (s_tot * s_tot - sq_tot)
    part_ref[0, 0, :] = (jnp.sum(e1_ref[...], 1) + jnp.sum(ei1_ref[...], 0)
                         + jnp.sum(fm2, 1))


def _kmid_body(h_ref, sp_ref, ssp_ref, g_ref, be_ref, W_ref, b_ref,
               out_ref, sp2_ref, ssp2_ref):
    s = jnp.sum(sp_ref[...], (0, 1))
    ss = jnp.sum(ssp_ref[...], (0, 1))
    m = s * (1.0 / B)
    v = ss * (1.0 / B) - m * m
    sc = g_ref[0, :] * lax.rsqrt(v + 1e-5)
    off = be_ref[0, :] - m * sc
    hin = jnp.maximum(h_ref[...] * sc + off, 0.0)
    h2 = jnp.dot(hin, W_ref[...], preferred_element_type=jnp.float32) + b_ref[...]
    out_ref[...] = h2
    sp2_ref[0, 0, :] = jnp.sum(h2, 0)
    ssp2_ref[0, 0, :] = jnp.sum(h2 * h2, 0)


def _k4_body(h_ref, sp_ref, ssp_ref, g_ref, be_ref, part_ref, bias_ref, out_ref):
    s = jnp.sum(sp_ref[...], (0, 1))
    ss = jnp.sum(ssp_ref[...], (0, 1))
    m = s * (1.0 / B)
    v = ss * (1.0 / B) - m * m
    sc = g_ref[0, :] * lax.rsqrt(v + 1e-5)
    off = be_ref[0, :] - m * sc
    hin = jnp.maximum(h_ref[...] * sc + off, 0.0)
    out_ref[0, 0, :] = part_ref[0, 0, :] + jnp.sum(hin, 1) + bias_ref[0, 0]


def _row(shape):
    return pl.BlockSpec(shape, lambda t: (t, 0))


def _full(shape):
    return pl.BlockSpec(shape, lambda t: tuple(0 for _ in shape))


def kernel(Xi, Xv, X_interest, w1, wi1, w2, wi2, bias, W1, b1, g1, be1,
           W2, b2, g2, be2, W3, b3, g3, be3):
    idx = Xi[:, :, 0].astype(jnp.int32)
    gidx_e = (idx + (jnp.arange(F, dtype=jnp.int32) * V)[None, :]).reshape(B * F)
    xvf = Xv.reshape(B * F)
    w2f = w2.reshape(F * V, D)
    w1f = w1.reshape(F * V)
    wi2f = wi2.reshape(NI * V, D)
    wi1f = wi1.reshape(NI * V)
    xt = jnp.transpose(X_interest.astype(jnp.int32), (1, 0, 2)).reshape(NI * B * H)

    e2s, e1s = _sc_gather_a(w2f, w1f, gidx_e, xvf)
    ei2f, ei1f = _sc_segsum_b(wi2f, wi1f, xt)

    e2m = e2s.reshape(B, F * D)
    e1m = e1s.reshape(B, F)
    ei2_3 = ei2f.reshape(NI, B, D)
    ei1_2 = ei1f.reshape(NI, B)
    Mmat = (jnp.arange(F * D, dtype=jnp.int32)[:, None] % D
            == jnp.arange(D, dtype=jnp.int32)[None, :]).astype(jnp.float32)

    f32 = jnp.float32
    h1, sp1, ssp1, part = pl.pallas_call(
        _k1_body,
        grid=(GT,),
        in_specs=[
            _row((BT, F * D)),
            pl.BlockSpec((NI, BT, D), lambda t: (0, t, 0)),
            pl.BlockSpec((NI, BT), lambda t: (0, t)),
            _row((BT, F)),
            _full((F * D, L1)),
            _full((1, L1)),
            _full((F * D, D)),
        ],
        out_specs=[
            _row((BT, L1)),
            pl.BlockSpec((1, 1, L1), lambda t: (t, 0, 0)),
            pl.BlockSpec((1, 1, L1), lambda t: (t, 0, 0)),
            pl.BlockSpec((1, 1, BT), lambda t: (t, 0, 0)),
        ],
        out_shape=[
            jax.ShapeDtypeStruct((B, L1), f32),
            jax.ShapeDtypeStruct((GT, 1, L1), f32),
            jax.ShapeDtypeStruct((GT, 1, L1), f32),
            jax.ShapeDtypeStruct((GT, 1, BT), f32),
        ],
    )(e2m, ei2_3, ei1_2, e1m, W1, b1.reshape(1, L1), Mmat)

    def mid(h, sp, ssp, g, be, W, b, L):
        return pl.pallas_call(
            _kmid_body,
            grid=(GT,),
            in_specs=[
                _row((BT, L)), _full((GT, 1, L)), _full((GT, 1, L)),
                _full((1, L)), _full((1, L)), _full((L, L)), _full((1, L)),
            ],
            out_specs=[
                _row((BT, L)),
                pl.BlockSpec((1, 1, L), lambda t: (t, 0, 0)),
                pl.BlockSpec((1, 1, L), lambda t: (t, 0, 0)),
            ],
            out_shape=[
                jax.ShapeDtypeStruct((B, L), f32),
                jax.ShapeDtypeStruct((GT, 1, L), f32),
                jax.ShapeDtypeStruct((GT, 1, L), f32),
            ],
        )(h, sp, ssp, g.reshape(1, L), be.reshape(1, L), W, b.reshape(1, L))

    h2, sp2, ssp2 = mid(h1, sp1, ssp1, g1, be1, W2, b2, L1)
    h3, sp3, ssp3 = mid(h2, sp2, ssp2, g2, be2, W3, b3, L2)

    tot = pl.pallas_call(
        _k4_body,
        grid=(GT,),
        in_specs=[
            _row((BT, L3)), _full((GT, 1, L3)), _full((GT, 1, L3)),
            _full((1, L3)), _full((1, L3)),
            pl.BlockSpec((1, 1, BT), lambda t: (t, 0, 0)), _full((1, 1)),
        ],
        out_specs=pl.BlockSpec((1, 1, BT), lambda t: (t, 0, 0)),
        out_shape=jax.ShapeDtypeStruct((GT, 1, BT), f32),
    )(h3, sp3, ssp3, g3.reshape(1, L3), be3.reshape(1, L3), part,
      bias.reshape(1, 1))

    return tot.reshape(B)
